# baseline (speedup 1.0000x reference)
import functools
import math

import numpy as np
import jax
import jax.numpy as jnp
from jax import lax
from jax.experimental import pallas as pl
from jax.experimental.pallas import tpu as pltpu
from jax.experimental.pallas import tpu_sc as plsc

_D = 256
_L = 4
_M = 8
_P = 4
_B = 8
_LQ = 300
_DH = _D // _M
_SPATIAL = ((64, 64), (32, 32), (16, 16), (8, 8))
_LIN = sum(h * w for h, w in _SPATIAL)
_STARTS = (0, 4096, 5120, 5376)
_LP = _L * _P
_NS = _LP * 4
_NW = 32
_QPW = (_B * _LQ) // _NW

_lvl = (np.arange(128) // _P) % _L
_WLF = np.asarray([_SPATIAL[l][1] for l in _lvl], np.float32)[None]
_HLF = np.asarray([_SPATIAL[l][0] for l in _lvl], np.float32)[None]
_WLI = _WLF.astype(np.int32)
_SLI = np.asarray([_STARTS[l] for l in _lvl], np.int32)[None]
_MLI = (np.arange(128) // _LP).astype(np.int32)[None]
_G2 = (np.arange(128)[:, None] // _LP == np.arange(128)[None, :] // _LP)
_G2 = _G2.astype(np.float32)

_CORNERS = ((0, 0), (1, 0), (0, 1), (1, 1))


def _prep_body(q_ref, rpx_ref, rpy_ref, wox_ref, woy_ref, wat_ref,
               box_ref, boy_ref, bat_ref, g2_ref,
               wlf_ref, hlf_ref, wli_ref, sli_ref, mli_ref, bvec_ref,
               idx_ref, wgt_ref):
    q = q_ref[...]
    sox = jnp.dot(q, wox_ref[...], preferred_element_type=jnp.float32) + box_ref[...]
    soy = jnp.dot(q, woy_ref[...], preferred_element_type=jnp.float32) + boy_ref[...]
    logits = jnp.dot(q, wat_ref[...], preferred_element_type=jnp.float32) + bat_ref[...]
    mx = jnp.max(logits, axis=-1, keepdims=True)
    e = jnp.exp(logits - mx)
    s = jnp.dot(e, g2_ref[...], preferred_element_type=jnp.float32)
    aw = e / s

    wlf = wlf_ref[...]
    hlf = hlf_ref[...]
    gx = rpx_ref[...] * wlf + sox - 0.5
    gy = rpy_ref[...] * hlf + soy - 0.5
    x0 = jnp.floor(gx)
    y0 = jnp.floor(gy)
    fx = gx - x0
    fy = gy - y0
    row_base = bvec_ref[...] + mli_ref[...]
    for n, (dx, dy) in enumerate(_CORNERS):
        xn = x0 + dx
        yn = y0 + dy
        wn = (fx if dx else 1.0 - fx) * (fy if dy else 1.0 - fy)
        valid = ((xn >= 0.0) & (xn <= wlf - 1.0)
                 & (yn >= 0.0) & (yn <= hlf - 1.0))
        xi = jnp.clip(xn, 0.0, wlf - 1.0).astype(jnp.int32)
        yi = jnp.clip(yn, 0.0, hlf - 1.0).astype(jnp.int32)
        lin = yi * wli_ref[...] + xi + sli_ref[...]
        idx_ref[:, pl.ds(n * 128, 128)] = lin * _M + row_base
        wgt_ref[:, pl.ds(n * 128, 128)] = aw * wn * valid.astype(jnp.float32)


_BVEC = ((np.arange(_B * _LQ) // _LQ) * (_LIN * _M)).astype(np.int32)[:, None]


def _prep(query, rpx, rpy, wox, woy, wat, box, boy, bat):
    g2 = jnp.asarray(_G2)
    nq = _B * _LQ
    return pl.pallas_call(
        _prep_body,
        out_shape=[
            jax.ShapeDtypeStruct((nq, 512), jnp.int32),
            jax.ShapeDtypeStruct((nq, 512), jnp.float32),
        ],
    )(query, rpx, rpy, wox, woy, wat, box, boy, bat, g2,
      jnp.asarray(_WLF), jnp.asarray(_HLF), jnp.asarray(_WLI),
      jnp.asarray(_SLI), jnp.asarray(_MLI), jnp.asarray(_BVEC))


def _matmul_bias_body(x_ref, w_ref, b_ref, o_ref):
    o_ref[...] = (jnp.dot(x_ref[...], w_ref[...],
                          preferred_element_type=jnp.float32)
                  + b_ref[...]).astype(o_ref.dtype)


def _matmul_bias(x, w_t, bias, block_rows, out_dtype=jnp.float32):
    n, k = x.shape
    m = w_t.shape[1]
    assert n % block_rows == 0
    return pl.pallas_call(
        _matmul_bias_body,
        grid=(n // block_rows,),
        in_specs=[
            pl.BlockSpec((block_rows, k), lambda i: (i, 0)),
            pl.BlockSpec((k, m), lambda i: (0, 0)),
            pl.BlockSpec((1, m), lambda i: (0, 0)),
        ],
        out_specs=pl.BlockSpec((block_rows, m), lambda i: (i, 0)),
        out_shape=jax.ShapeDtypeStruct((n, m), out_dtype),
    )(x, w_t, bias)


def _sc_body(table_ref, idxs_ref, wgts_ref, out_ref,
             idx_all, wgt_v, rows_v, out_v, s_i, sg0, sg1, sg2, sw0, sw1, sw2):
    wid = lax.axis_index("s") * 2 + lax.axis_index("c")
    pltpu.async_copy(idxs_ref.at[wid], idx_all, s_i).wait()

    def issue(buf, c, sg, sw):
        for n in range(4):
            pltpu.async_copy(table_ref.at[idx_all.at[c, n]],
                             rows_v.at[buf, pl.ds(n * 128, 128)], sg)
        pltpu.async_copy(wgts_ref.at[wid, c], wgt_v.at[buf], sw)

    def wait_query(buf, c, sg, sw):
        for n in range(4):
            pltpu.make_async_copy(table_ref.at[idx_all.at[c, n]],
                                  rows_v.at[buf, pl.ds(n * 128, 128)],
                                  sg).wait()
        pltpu.make_async_copy(wgts_ref.at[wid, c], wgt_v.at[buf], sw).wait()

    def compute(buf, c):
        def head(m):
            acc0 = jnp.zeros((16,), jnp.float32)
            acc1 = jnp.zeros((16,), jnp.float32)
            for n in range(4):
                woff = n * 128 + m * 16
                wv = wgt_v[buf, pl.ds(woff, 16)]
                for j in range(16):
                    w = wv[j]
                    row = rows_v[buf, woff + j, :]
                    ev, od = plsc.unpack(row, format=plsc.PackFormat.INTERLEAVED,
                                         preferred_element_type=jnp.float32)
                    acc0 = acc0 + w * ev
                    acc1 = acc1 + w * od
            out_v[c, m, pl.ds(0, 16)] = acc0
            out_v[c, m, pl.ds(16, 16)] = acc1

        for m in range(_M):
            head(m)

    sgs = (sg0, sg1, sg2)
    sws = (sw0, sw1, sw2)
    last = _QPW - 1
    issue(0, 0, sg0, sw0)
    issue(1, 1, sg1, sw1)

    def it(k, carry):
        c = 3 * k
        issue(2, c + 2, sgs[2], sws[2])
        for b in range(3):
            wait_query(b, c + b, sgs[b], sws[b])
            compute(b, c + b)
            if b < 2:
                nxt = jnp.minimum(c + 3 + b, last)
                issue(b, nxt, sgs[b], sws[b])
        return carry

    lax.fori_loop(0, _QPW // 3, it, 0)
    wait_query(0, last, sg0, sw0)
    wait_query(1, last, sg1, sw1)
    pltpu.sync_copy(out_v, out_ref.at[wid])


def _sc_gather_combine(table, idxs, wgts):
    mesh = plsc.VectorSubcoreMesh(core_axis_name="c", subcore_axis_name="s",
                                  num_cores=2, num_subcores=16)
    kern = pl.kernel(
        _sc_body,
        out_type=jax.ShapeDtypeStruct((_NW, _QPW, _M, _DH), jnp.float32),
        mesh=mesh,
        scratch_types=[
            pltpu.VMEM((_QPW, 4, 128), jnp.int32),
            pltpu.VMEM((3, 512), jnp.float32),
            pltpu.VMEM((3, 512, _DH), jnp.bfloat16),
            pltpu.VMEM((_QPW, _M, _DH), jnp.float32),
            pltpu.SemaphoreType.DMA,
            pltpu.SemaphoreType.DMA,
            pltpu.SemaphoreType.DMA,
            pltpu.SemaphoreType.DMA,
            pltpu.SemaphoreType.DMA,
            pltpu.SemaphoreType.DMA,
            pltpu.SemaphoreType.DMA,
        ],
        compiler_params=pltpu.CompilerParams(use_tc_tiling_on_sc=False,
                                             needs_layout_passes=False),
    )
    return kern(table, idxs, wgts)


def kernel(query, reference_points, input_flatten, input_spatial_shapes,
           input_level_start_index, W_off, b_off, W_attn, b_attn,
           W_val, b_val, W_out, b_out):
    wox = W_off[0::2].T
    woy = W_off[1::2].T
    box = b_off[0::2][None]
    boy = b_off[1::2][None]
    _aperm = np.arange(128).reshape(_M, _L, _P).transpose(0, 2, 1).reshape(-1)
    wat = W_attn[_aperm].T
    bat = b_attn[_aperm][None]
    rpx = jnp.broadcast_to(
        reference_points[:, :, None, :, None, 0], (_B, _LQ, _M, _L, _P)
    ).reshape(_B * _LQ, 128)
    rpy = jnp.broadcast_to(
        reference_points[:, :, None, :, None, 1], (_B, _LQ, _M, _L, _P)
    ).reshape(_B * _LQ, 128)

    idx4, wgt4 = _prep(query.reshape(_B * _LQ, _D), rpx, rpy,
                       wox, woy, wat, box, boy, bat)
    idxs = idx4.reshape(_NW, _QPW, 4, 128)
    wgts = wgt4.reshape(_NW, _QPW, 512)

    value = _matmul_bias(input_flatten.reshape(_B * _LIN, _D), W_val.T,
                         b_val[None], 512, out_dtype=jnp.bfloat16)
    table = value.reshape(_B * _LIN * _M, _DH)

    attn = _sc_gather_combine(table, idxs, wgts)
    attn = attn.reshape(_B * _LQ, _D)

    ch = np.arange(_DH)
    src = (np.arange(_M)[:, None] * _DH
           + np.concatenate([ch[0::2], ch[1::2]])[None, :]).reshape(-1)
    out = _matmul_bias(attn, W_out.T[src], b_out[None], 480)
    return out.reshape(_B, _LQ, _D)


# speedup vs baseline: 1.4685x; 1.4685x over previous
import functools
import math

import numpy as np
import jax
import jax.numpy as jnp
from jax import lax
from jax.experimental import pallas as pl
from jax.experimental.pallas import tpu as pltpu
from jax.experimental.pallas import tpu_sc as plsc

_D = 256
_L = 4
_M = 8
_P = 4
_B = 8
_LQ = 300
_DH = _D // _M
_SPATIAL = ((64, 64), (32, 32), (16, 16), (8, 8))
_LIN = sum(h * w for h, w in _SPATIAL)
_STARTS = (0, 4096, 5120, 5376)
_LP = _L * _P
_NS = _LP * 4
_NW = 32
_QPW = (_B * _LQ) // _NW

_lvl = (np.arange(128) // _P) % _L
_WLF = np.asarray([_SPATIAL[l][1] for l in _lvl], np.float32)[None]
_HLF = np.asarray([_SPATIAL[l][0] for l in _lvl], np.float32)[None]
_WLI = _WLF.astype(np.int32)
_SLI = np.asarray([_STARTS[l] for l in _lvl], np.int32)[None]
_MLI = (np.arange(128) // _LP).astype(np.int32)[None]
_G2 = (np.arange(128)[:, None] // _LP == np.arange(128)[None, :] // _LP)
_G2 = _G2.astype(np.float32)

_CORNERS = ((0, 0), (1, 0), (0, 1), (1, 1))


def _prep_body(q_ref, rpx_ref, rpy_ref, wox_ref, woy_ref, wat_ref,
               box_ref, boy_ref, bat_ref, g2_ref,
               wlf_ref, hlf_ref, wli_ref, sli_ref, mli_ref, bvec_ref,
               idx_ref, wgt_ref):
    q = q_ref[...]
    sox = jnp.dot(q, wox_ref[...], preferred_element_type=jnp.float32) + box_ref[...]
    soy = jnp.dot(q, woy_ref[...], preferred_element_type=jnp.float32) + boy_ref[...]
    logits = jnp.dot(q, wat_ref[...], preferred_element_type=jnp.float32) + bat_ref[...]
    mx = jnp.max(logits, axis=-1, keepdims=True)
    e = jnp.exp(logits - mx)
    s = jnp.dot(e, g2_ref[...], preferred_element_type=jnp.float32)
    aw = e / s

    wlf = wlf_ref[...]
    hlf = hlf_ref[...]
    gx = rpx_ref[...] * wlf + sox - 0.5
    gy = rpy_ref[...] * hlf + soy - 0.5
    x0 = jnp.floor(gx)
    y0 = jnp.floor(gy)
    fx = gx - x0
    fy = gy - y0
    row_base = bvec_ref[...] + mli_ref[...]
    for n, (dx, dy) in enumerate(_CORNERS):
        xn = x0 + dx
        yn = y0 + dy
        wn = (fx if dx else 1.0 - fx) * (fy if dy else 1.0 - fy)
        valid = ((xn >= 0.0) & (xn <= wlf - 1.0)
                 & (yn >= 0.0) & (yn <= hlf - 1.0))
        xi = jnp.clip(xn, 0.0, wlf - 1.0).astype(jnp.int32)
        yi = jnp.clip(yn, 0.0, hlf - 1.0).astype(jnp.int32)
        lin = yi * wli_ref[...] + xi + sli_ref[...]
        idx_ref[:, pl.ds(n * 128, 128)] = lin * _M + row_base
        wgt_ref[:, pl.ds(n * 128, 128)] = aw * wn * valid.astype(jnp.float32)


_BVEC = ((np.arange(_B * _LQ) // _LQ) * (_LIN * _M)).astype(np.int32)[:, None]


def _prep(query, rpx, rpy, wox, woy, wat, box, boy, bat):
    g2 = jnp.asarray(_G2)
    nq = _B * _LQ
    return pl.pallas_call(
        _prep_body,
        out_shape=[
            jax.ShapeDtypeStruct((nq, 512), jnp.int32),
            jax.ShapeDtypeStruct((nq, 512), jnp.float32),
        ],
    )(query, rpx, rpy, wox, woy, wat, box, boy, bat, g2,
      jnp.asarray(_WLF), jnp.asarray(_HLF), jnp.asarray(_WLI),
      jnp.asarray(_SLI), jnp.asarray(_MLI), jnp.asarray(_BVEC))


def _matmul_bias_body(x_ref, w_ref, b_ref, o_ref):
    o_ref[...] = (jnp.dot(x_ref[...], w_ref[...],
                          preferred_element_type=jnp.float32)
                  + b_ref[...]).astype(o_ref.dtype)


def _matmul_bias(x, w_t, bias, block_rows, out_dtype=jnp.float32):
    n, k = x.shape
    m = w_t.shape[1]
    assert n % block_rows == 0
    return pl.pallas_call(
        _matmul_bias_body,
        grid=(n // block_rows,),
        in_specs=[
            pl.BlockSpec((block_rows, k), lambda i: (i, 0)),
            pl.BlockSpec((k, m), lambda i: (0, 0)),
            pl.BlockSpec((1, m), lambda i: (0, 0)),
        ],
        out_specs=pl.BlockSpec((block_rows, m), lambda i: (i, 0)),
        out_shape=jax.ShapeDtypeStruct((n, m), out_dtype),
    )(x, w_t, bias)


def _sc_body(table_ref, idxs_ref, wgts_ref, out_ref,
             idx_all, wgt_v, rows_v, out_v, s_i, sg0, sg1, sg2, sw0, sw1, sw2):
    wid = lax.axis_index("s") * 2 + lax.axis_index("c")
    pltpu.async_copy(idxs_ref.at[wid], idx_all, s_i).wait()

    def issue(buf, c, sg, sw):
        pltpu.async_copy(table_ref.at[idx_all.at[c]], rows_v.at[buf], sg)
        pltpu.async_copy(wgts_ref.at[wid, c], wgt_v.at[buf], sw)

    def wait_query(buf, c, sg, sw):
        pltpu.make_async_copy(table_ref.at[idx_all.at[c]], rows_v.at[buf],
                              sg).wait()
        pltpu.make_async_copy(wgts_ref.at[wid, c], wgt_v.at[buf], sw).wait()

    def compute(buf, c):
        def head(m, carry):
            acc0 = jnp.zeros((16,), jnp.float32)
            acc1 = jnp.zeros((16,), jnp.float32)
            for n in range(4):
                woff = n * 128 + m * 16
                wv = wgt_v[buf, pl.ds(woff, 16)]
                for j in range(16):
                    w = wv[j]
                    row = rows_v[buf, woff + j, :]
                    ev, od = plsc.unpack(row, format=plsc.PackFormat.INTERLEAVED,
                                         preferred_element_type=jnp.float32)
                    acc0 = acc0 + w * ev
                    acc1 = acc1 + w * od
            out_v[c, m, pl.ds(0, 16)] = acc0
            out_v[c, m, pl.ds(16, 16)] = acc1
            return carry

        lax.fori_loop(0, _M, head, 0)

    sgs = (sg0, sg1, sg2)
    sws = (sw0, sw1, sw2)
    last = _QPW - 1
    issue(0, 0, sg0, sw0)
    issue(1, 1, sg1, sw1)

    def it(k, carry):
        c = 3 * k
        issue(2, c + 2, sgs[2], sws[2])
        for b in range(3):
            wait_query(b, c + b, sgs[b], sws[b])
            compute(b, c + b)
            if b < 2:
                nxt = jnp.minimum(c + 3 + b, last)
                issue(b, nxt, sgs[b], sws[b])
        return carry

    lax.fori_loop(0, _QPW // 3, it, 0)
    wait_query(0, last, sg0, sw0)
    wait_query(1, last, sg1, sw1)
    pltpu.sync_copy(out_v, out_ref.at[wid])


def _sc_gather_combine(table, idxs, wgts):
    mesh = plsc.VectorSubcoreMesh(core_axis_name="c", subcore_axis_name="s",
                                  num_cores=2, num_subcores=16)
    kern = pl.kernel(
        _sc_body,
        out_type=jax.ShapeDtypeStruct((_NW, _QPW, _M, _DH), jnp.float32),
        mesh=mesh,
        scratch_types=[
            pltpu.VMEM((_QPW, 512), jnp.int32),
            pltpu.VMEM((3, 512), jnp.float32),
            pltpu.VMEM((3, 512, _DH), jnp.bfloat16),
            pltpu.VMEM((_QPW, _M, _DH), jnp.float32),
            pltpu.SemaphoreType.DMA,
            pltpu.SemaphoreType.DMA,
            pltpu.SemaphoreType.DMA,
            pltpu.SemaphoreType.DMA,
            pltpu.SemaphoreType.DMA,
            pltpu.SemaphoreType.DMA,
            pltpu.SemaphoreType.DMA,
        ],
        compiler_params=pltpu.CompilerParams(use_tc_tiling_on_sc=False,
                                             needs_layout_passes=False),
    )
    return kern(table, idxs, wgts)


def kernel(query, reference_points, input_flatten, input_spatial_shapes,
           input_level_start_index, W_off, b_off, W_attn, b_attn,
           W_val, b_val, W_out, b_out):
    wox = W_off[0::2].T
    woy = W_off[1::2].T
    box = b_off[0::2][None]
    boy = b_off[1::2][None]
    _aperm = np.arange(128).reshape(_M, _L, _P).transpose(0, 2, 1).reshape(-1)
    wat = W_attn[_aperm].T
    bat = b_attn[_aperm][None]
    rpx = jnp.broadcast_to(
        reference_points[:, :, None, :, None, 0], (_B, _LQ, _M, _L, _P)
    ).reshape(_B * _LQ, 128)
    rpy = jnp.broadcast_to(
        reference_points[:, :, None, :, None, 1], (_B, _LQ, _M, _L, _P)
    ).reshape(_B * _LQ, 128)

    idx4, wgt4 = _prep(query.reshape(_B * _LQ, _D), rpx, rpy,
                       wox, woy, wat, box, boy, bat)
    idxs = idx4.reshape(_NW, _QPW, 512)
    wgts = wgt4.reshape(_NW, _QPW, 512)

    value = _matmul_bias(input_flatten.reshape(_B * _LIN, _D), W_val.T,
                         b_val[None], 512, out_dtype=jnp.bfloat16)
    table = value.reshape(_B * _LIN * _M, _DH)

    attn = _sc_gather_combine(table, idxs, wgts)
    attn = attn.reshape(_B * _LQ, _D)

    ch = np.arange(_DH)
    src = (np.arange(_M)[:, None] * _DH
           + np.concatenate([ch[0::2], ch[1::2]])[None, :]).reshape(-1)
    out = _matmul_bias(attn, W_out.T[src], b_out[None], 480)
    return out.reshape(_B, _LQ, _D)
